# baseline (speedup 1.0000x reference)
import numpy as np
import jax
import jax.numpy as jnp
from jax import lax
from jax.experimental import pallas as pl
from jax.experimental.pallas import tpu as pltpu
from jax.experimental.pallas import tpu_sc as plsc

_EPS = 1e-3
_NC, _NS = 2, 16


def _round_up(x, m):
    return (x + m - 1) // m * m


def _prep_tc(h2, dnr2, kh, scal, ei2, lg2, n, e, ep, tp):
    tch = h2.shape[0]
    na, hid = kh.shape[1], kh.shape[2]
    er = ei2.shape[1]
    epr = ep // 128
    tpr = tp // 128
    ta = (tch // 8) * 8
    ea = (er // 8) * 8

    def body(h_ref, d_ref, k_ref, s_ref, ei_ref, lg_ref,
             sp_ref, w_ref, eip_ref, lgp_ref):
        x = jnp.clip(h_ref[...], -_EPS, _EPS)
        theta = jnp.float32(np.pi / 2) - x - (x * x * x) * jnp.float32(1.0 / 6.0)
        d2 = d_ref[...] * d_ref[...]
        for hd in range(2):
            av = s_ref[0, hd]
            bv = s_ref[1, hd]
            cv = s_ref[2, hd]
            dv = s_ref[3, hd]
            ang = ((jnp.cos(av * theta + bv) + 1.0) * 0.5) ** cv
            rad = jnp.exp(-dv * d2)
            spv = ang * rad
            sp_ref[hd, pl.ds(0, ta)] = spv[:ta]
            sp_ref[hd, pl.ds(ta, tpr - ta)] = jnp.concatenate(
                [spv[ta:], jnp.zeros((tpr - tch, 128), jnp.float32)], axis=0)
            k = k_ref[hd]
            w_ref[hd] = lax.dot_general(
                k, k, (((1,), (1,)), ((), ())),
                preferred_element_type=jnp.float32)
        for hd, padv in ((0, n), (1, 0)):
            eip_ref[hd, pl.ds(0, ea)] = ei_ref[hd, pl.ds(0, ea)]
            eip_ref[hd, pl.ds(ea, epr - ea)] = jnp.concatenate(
                [ei_ref[hd, pl.ds(ea, er - ea)],
                 jnp.full((epr - er, 128), padv, jnp.int32)], axis=0)
        for hd, padv in ((0, e), (1, 0)):
            lgp_ref[hd, pl.ds(0, ta)] = lg_ref[hd, pl.ds(0, ta)]
            lgp_ref[hd, pl.ds(ta, tpr - ta)] = jnp.concatenate(
                [lg_ref[hd, pl.ds(ta, tch - ta)],
                 jnp.full((tpr - tch, 128), padv, jnp.int32)], axis=0)

    return pl.pallas_call(
        body,
        in_specs=[
            pl.BlockSpec(memory_space=pltpu.VMEM),
            pl.BlockSpec(memory_space=pltpu.VMEM),
            pl.BlockSpec(memory_space=pltpu.VMEM),
            pl.BlockSpec(memory_space=pltpu.SMEM),
            pl.BlockSpec(memory_space=pltpu.VMEM),
            pl.BlockSpec(memory_space=pltpu.VMEM),
        ],
        out_shape=(
            jax.ShapeDtypeStruct((2, tpr, 128), jnp.float32),
            jax.ShapeDtypeStruct((2, na, na), jnp.float32),
            jax.ShapeDtypeStruct((2, epr, 128), jnp.int32),
            jax.ShapeDtypeStruct((2, tpr, 128), jnp.int32),
        ),
    )(h2, dnr2, kh, scal, ei2, lg2)


def _phase1_sc(an, src_p, dst_p, lsrc2, ldst2, wflat,
               nax, na, e, ep, tp):
    ept = ep // _NS
    tt = tp // (_NC * _NS)
    nch = tt // 1024
    mesh = plsc.VectorSubcoreMesh(core_axis_name="c", subcore_axis_name="s")

    def body(an_h, src_h, dst_h, lgp_h, sp_h, w_h,
             sv_h, cnt_h,
             an_v, w_v, srcbufA, srcbufB, idsbufA, idsbufB,
             lsA, lsB, ldA, ldB, s0A, s0B, s1A, s1B,
             kiA, kiB, kjA, kjB, v0A, v0B, v1A, v1B,
             ones_v, zbuf, lsemA, lsemB, gsemA, gsemB, ssemA, ssemB, zsem,
             ks_sh, kd_sh, s0_sh, s1_sh, c_sh):
        c = lax.axis_index("c")
        s = lax.axis_index("s")
        lsems = [lsemA, lsemB]
        ssems = [ssemA, ssemB]
        pltpu.sync_copy(an_h, an_v)
        pltpu.sync_copy(w_h, w_v)

        def zz(k, _):
            zbuf[pl.ds(k * 16, 16)] = jnp.zeros((16,), jnp.float32)
            return _

        lax.fori_loop(0, 128, zz, None)

        def oo(k, _):
            ones_v[pl.ds(k * 16, 16)] = jnp.ones((16,), jnp.float32)
            return _

        lax.fori_loop(0, 8, oo, None)
        zdescs = []
        for k in range(ep // _NS // 2048):
            off = pl.multiple_of(s * (ep // _NS) + k * 2048, 2048)
            for sh in (s0_sh, s1_sh, c_sh):
                zdescs.append(
                    pltpu.async_copy(zbuf, sh.at[pl.ds(off, 2048)], zsem))

        sbufs = [srcbufA, srcbufB]
        ibufs = [idsbufA, idsbufB]
        steps = ([(src_h, ks_sh, bi) for bi in range(ept // 2048)]
                 + [(dst_h, kd_sh, bi) for bi in range(ept // 2048)])
        nst = len(steps)

        def s1_load(i):
            eh, _, bi = steps[i]
            ebase = pl.multiple_of(s * ept + bi * 2048, 2048)
            return pltpu.async_copy(eh.at[pl.ds(ebase, 2048)],
                                    sbufs[i % 2], lsems[i % 2])

        ldd = {0: s1_load(0)}
        std = {}
        for i in range(nst):
            p = i % 2
            if i >= 2:
                std[i - 2].wait()
            if i + 1 < nst:
                ldd[i + 1] = s1_load(i + 1)
            ldd[i].wait()

            def gg(g, _, p=p):
                for u in range(4):
                    sl = pl.ds((g * 4 + u) * 16, 16)
                    ibufs[p][sl] = plsc.load_gather(an_v, [sbufs[p][sl]])
                return _

            lax.fori_loop(0, 32, gg, None)
            _, sh, bi = steps[i]
            ebase = pl.multiple_of(s * ept + bi * 2048, 2048)
            std[i] = pltpu.async_copy(ibufs[p], sh.at[pl.ds(ebase, 2048)],
                                      ssems[p])
        std[nst - 2].wait()
        std[nst - 1].wait()
        for dd in zdescs:
            dd.wait()
        plsc.subcore_barrier()

        ls = [lsA, lsB]
        ld = [ldA, ldB]
        s0b = [s0A, s0B]
        s1b = [s1A, s1B]
        ki = [kiA, kiB]
        kj = [kjA, kjB]
        v0b = [v0A, v0B]
        v1b = [v1A, v1B]
        lsems = [lsemA, lsemB]
        gsems = [gsemA, gsemB]
        ssems = [ssemA, ssemB]
        rowbase = c * (tp // 2 // 128) + s * (tt // 128)

        def issue_loads(ch):
            p = ch % 2
            rb = pl.multiple_of(rowbase + ch * 8, 8)
            return [
                pltpu.async_copy(lgp_h.at[0].at[pl.ds(rb, 8)],
                                 ls[p], lsems[p]),
                pltpu.async_copy(lgp_h.at[1].at[pl.ds(rb, 8)],
                                 ld[p], lsems[p]),
                pltpu.async_copy(sp_h.at[0].at[pl.ds(rb, 8)],
                                 s0b[p], lsems[p]),
                pltpu.async_copy(sp_h.at[1].at[pl.ds(rb, 8)],
                                 s1b[p], lsems[p]),
            ]

        def issue_gathers(p, si):
            gp = si % 2
            return [
                pltpu.async_copy(ks_sh.at[ls[p].at[si]], ki[gp], gsems[gp]),
                pltpu.async_copy(kd_sh.at[ld[p].at[si]], kj[gp], gsems[gp]),
            ]

        loads = {0: issue_loads(0)}
        scats = {}
        for ch in range(nch):
            p = ch % 2
            if ch >= 1:
                for dd in scats[ch - 1]:
                    dd.wait()
            if ch + 1 < nch:
                loads[ch + 1] = issue_loads(ch + 1)
            for dd in loads[ch]:
                dd.wait()
            gcur = issue_gathers(p, 0)
            pend = []
            for si in range(8):
                gp = si % 2
                if si < 7:
                    gnext = issue_gathers(p, si + 1)
                for dd in gcur:
                    dd.wait()

                def grp(g, _, si=si, gp=gp, p=p):
                    i = ki[gp][pl.ds(g * 16, 16)]
                    j = kj[gp][pl.ds(g * 16, 16)]
                    fidx = i * na + j
                    w0 = plsc.load_gather(w_v, [fidx])
                    w1 = plsc.load_gather(w_v, [fidx + na * na])
                    v0b[p][si, pl.ds(g * 16, 16)] = (
                        w0 * s0b[p][si, pl.ds(g * 16, 16)])
                    v1b[p][si, pl.ds(g * 16, 16)] = (
                        w1 * s1b[p][si, pl.ds(g * 16, 16)])
                    return _

                lax.fori_loop(0, 8, grp, None)
                pend.append(pltpu.async_copy(
                    v0b[p].at[si], s0_sh.at[ls[p].at[si]], ssems[p],
                    add=True))
                pend.append(pltpu.async_copy(
                    v1b[p].at[si], s1_sh.at[ls[p].at[si]], ssems[p],
                    add=True))
                pend.append(pltpu.async_copy(
                    ones_v, c_sh.at[ls[p].at[si]], ssems[p], add=True))
                gcur = gnext
            scats[ch] = pend
        for dd in scats[nch - 1]:
            dd.wait()
        plsc.subcore_barrier()

        odescs = []
        for k in range(ep // _NS // 1280):
            off = pl.multiple_of(s * (ep // _NS) + k * 1280, 1280)
            sl = pl.ds(off, 1280)
            odescs.append(pltpu.async_copy(
                s0_sh.at[sl], sv_h.at[c].at[0].at[sl], ssemA))
            odescs.append(pltpu.async_copy(
                s1_sh.at[sl], sv_h.at[c].at[1].at[sl], ssemB))
            odescs.append(pltpu.async_copy(
                c_sh.at[sl], cnt_h.at[c].at[sl], zsem))
        for dd in odescs:
            dd.wait()

    return pl.kernel(
        body,
        out_type=(
            jax.ShapeDtypeStruct((_NC, 2, ep), jnp.float32),
            jax.ShapeDtypeStruct((_NC, ep), jnp.float32),
        ),
        mesh=mesh,
        compiler_params=pltpu.CompilerParams(needs_layout_passes=False),
        scratch_types=[
            pltpu.VMEM((nax,), jnp.int32),
            pltpu.VMEM((2 * na * na,), jnp.float32),
            pltpu.VMEM((2048,), jnp.int32),
            pltpu.VMEM((2048,), jnp.int32),
            pltpu.VMEM((2048,), jnp.int32),
            pltpu.VMEM((2048,), jnp.int32),
            pltpu.VMEM((8, 128), jnp.int32),
            pltpu.VMEM((8, 128), jnp.int32),
            pltpu.VMEM((8, 128), jnp.int32),
            pltpu.VMEM((8, 128), jnp.int32),
            pltpu.VMEM((8, 128), jnp.float32),
            pltpu.VMEM((8, 128), jnp.float32),
            pltpu.VMEM((8, 128), jnp.float32),
            pltpu.VMEM((8, 128), jnp.float32),
            pltpu.VMEM((128,), jnp.int32),
            pltpu.VMEM((128,), jnp.int32),
            pltpu.VMEM((128,), jnp.int32),
            pltpu.VMEM((128,), jnp.int32),
            pltpu.VMEM((8, 128), jnp.float32),
            pltpu.VMEM((8, 128), jnp.float32),
            pltpu.VMEM((8, 128), jnp.float32),
            pltpu.VMEM((8, 128), jnp.float32),
            pltpu.VMEM((128,), jnp.float32),
            pltpu.VMEM((2048,), jnp.float32),
            pltpu.SemaphoreType.DMA,
            pltpu.SemaphoreType.DMA,
            pltpu.SemaphoreType.DMA,
            pltpu.SemaphoreType.DMA,
            pltpu.SemaphoreType.DMA,
            pltpu.SemaphoreType.DMA,
            pltpu.SemaphoreType.DMA,
            pltpu.VMEM_SHARED((ep,), jnp.int32),
            pltpu.VMEM_SHARED((ep,), jnp.int32),
            pltpu.VMEM_SHARED((ep,), jnp.float32),
            pltpu.VMEM_SHARED((ep,), jnp.float32),
            pltpu.VMEM_SHARED((ep,), jnp.float32),
        ],
    )(an, src_p, dst_p, lsrc2, ldst2, wflat)


def _phase2_sc(an, eip, sv, cp, nax, np2, nap, ep):
    qn = nap * np2
    et = ep // _NS
    nch = et // 2048
    qt = qn // _NS
    ct = np2 // _NS
    mesh = plsc.VectorSubcoreMesh(core_axis_name="c", subcore_axis_name="s")

    def body(an_h, eip_h, sv_h, cp_h, q_h, cnt_h,
             an_v, srA, srB, dsA, dsB,
             vaA, vaB, vbA, vbB, caA, caB, cbA, cbB,
             qiA, qiB, cvA, cvB,
             ones_v, zq, lsemA, lsemB, ssemA, ssemB, zsem, q_sp, cnt_sp):
        c = lax.axis_index("c")
        s = lax.axis_index("s")
        pltpu.sync_copy(an_h, an_v)

        def zz(k, _):
            zq[pl.ds(k * 16, 16)] = jnp.zeros((16,), jnp.float32)
            return _

        lax.fori_loop(0, 160, zz, None)

        def oo(k, _):
            ones_v[pl.ds(k * 16, 16)] = jnp.ones((16,), jnp.float32)
            return _

        lax.fori_loop(0, 8, oo, None)
        zdescs = [pltpu.async_copy(zq.at[pl.ds(0, ct)],
                                   cnt_sp.at[pl.ds(s * ct, ct)], zsem)]
        for k in range(qt // 2560):
            off = pl.multiple_of(s * qt + k * 2560, 2560)
            zdescs.append(pltpu.async_copy(zq, q_sp.at[pl.ds(off, 2560)],
                                           zsem))
        for dd in zdescs:
            dd.wait()
        plsc.subcore_barrier()

        sr = [srA, srB]
        ds2 = [dsA, dsB]
        va = [vaA, vaB]
        vb = [vbA, vbB]
        ca = [caA, caB]
        cb = [cbA, cbB]
        qi2 = [qiA, qiB]
        cv2 = [cvA, cvB]
        lsems = [lsemA, lsemB]
        ssems = [ssemA, ssemB]
        estart = s * et
        rstart = s * (et // 128)

        def issue_loads(ch):
            p = ch % 2
            eb = pl.multiple_of(estart + ch * 2048, 2048)
            rb = pl.multiple_of(rstart + ch * 16, 16)
            return [
                pltpu.async_copy(eip_h.at[0].at[pl.ds(rb, 16)],
                                 sr[p], lsems[p]),
                pltpu.async_copy(eip_h.at[1].at[pl.ds(rb, 16)],
                                 ds2[p], lsems[p]),
                pltpu.async_copy(sv_h.at[0].at[c].at[pl.ds(eb, 2048)],
                                 va[p], lsems[p]),
                pltpu.async_copy(sv_h.at[1].at[c].at[pl.ds(eb, 2048)],
                                 vb[p], lsems[p]),
                pltpu.async_copy(cp_h.at[0].at[pl.ds(eb, 2048)],
                                 ca[p], lsems[p]),
                pltpu.async_copy(cp_h.at[1].at[pl.ds(eb, 2048)],
                                 cb[p], lsems[p]),
            ]

        loads = {0: issue_loads(0)}
        scats = {}
        for ch in range(nch):
            p = ch % 2
            if ch >= 1:
                for dd in scats[ch - 1]:
                    dd.wait()
            if ch + 1 < nch:
                loads[ch + 1] = issue_loads(ch + 1)
            for dd in loads[ch]:
                dd.wait()

            def grp(g, _, p=p):
                gd = g // 8
                off = (g % 8) * 16
                sv16 = sr[p][gd, pl.ds(off, 16)]
                dv = ds2[p][gd, pl.ds(off, 16)]
                kd = plsc.load_gather(an_v, [dv])
                sl = pl.ds(g * 16, 16)
                coef = (va[p][sl] + vb[p][sl]) / jnp.maximum(
                    ca[p][sl] + cb[p][sl], 1.0)
                qi2[p][gd, pl.ds(off, 16)] = kd * np2 + sv16
                cv2[p][gd, pl.ds(off, 16)] = coef
                return _

            lax.fori_loop(0, 128, grp, None)
            pend = []
            for k in range(16):
                pend.append(pltpu.async_copy(
                    cv2[p].at[k], q_sp.at[qi2[p].at[k]], ssems[p], add=True))
                pend.append(pltpu.async_copy(
                    ones_v, cnt_sp.at[sr[p].at[k]], ssems[p], add=True))
            scats[ch] = pend

        for dd in scats[nch - 1]:
            dd.wait()
        plsc.subcore_barrier()

        odescs = []
        for k in range(qt // 2560):
            off = pl.multiple_of(s * qt + k * 2560, 2560)
            odescs.append(pltpu.async_copy(
                q_sp.at[pl.ds(off, 2560)], q_h.at[c].at[pl.ds(off, 2560)],
                ssemA if k % 2 == 0 else ssemB))
        for dd in odescs:
            dd.wait()

        @pl.when(c == 0)
        def _cout():
            pltpu.sync_copy(cnt_sp.at[pl.ds(s * ct, ct)],
                            cnt_h.at[pl.ds(s * ct, ct)])

    return pl.kernel(
        body,
        out_type=(
            jax.ShapeDtypeStruct((_NC, qn), jnp.float32),
            jax.ShapeDtypeStruct((np2,), jnp.float32),
        ),
        mesh=mesh,
        compiler_params=pltpu.CompilerParams(needs_layout_passes=False),
        scratch_types=[
            pltpu.VMEM((nax,), jnp.int32),
            pltpu.VMEM((16, 128), jnp.int32),
            pltpu.VMEM((16, 128), jnp.int32),
            pltpu.VMEM((16, 128), jnp.int32),
            pltpu.VMEM((16, 128), jnp.int32),
            pltpu.VMEM((2048,), jnp.float32),
            pltpu.VMEM((2048,), jnp.float32),
            pltpu.VMEM((2048,), jnp.float32),
            pltpu.VMEM((2048,), jnp.float32),
            pltpu.VMEM((2048,), jnp.float32),
            pltpu.VMEM((2048,), jnp.float32),
            pltpu.VMEM((2048,), jnp.float32),
            pltpu.VMEM((2048,), jnp.float32),
            pltpu.VMEM((16, 128), jnp.int32),
            pltpu.VMEM((16, 128), jnp.int32),
            pltpu.VMEM((16, 128), jnp.float32),
            pltpu.VMEM((16, 128), jnp.float32),
            pltpu.VMEM((128,), jnp.float32),
            pltpu.VMEM((2560,), jnp.float32),
            pltpu.SemaphoreType.DMA,
            pltpu.SemaphoreType.DMA,
            pltpu.SemaphoreType.DMA,
            pltpu.SemaphoreType.DMA,
            pltpu.SemaphoreType.DMA,
            pltpu.VMEM_SHARED((qn,), jnp.float32),
            pltpu.VMEM_SHARED((np2,), jnp.float32),
        ],
    )(an, eip, sv, cp)


def _final_tc(qf, v0, v1, cnt2, nap, np2):
    outf = v0.shape[1]

    def body(q_ref, v0_ref, v1_ref, c_ref, o_ref):
        q0 = q_ref[0].reshape(nap, np2)
        q1 = q_ref[1].reshape(nap, np2)
        acc = lax.dot_general(q0, v0_ref[...], (((0,), (0,)), ((), ())),
                              preferred_element_type=jnp.float32)
        acc = acc + lax.dot_general(q1, v1_ref[...], (((0,), (0,)), ((), ())),
                                    preferred_element_type=jnp.float32)
        o_ref[...] = acc / jnp.maximum(c_ref[...], 1.0)

    return pl.pallas_call(
        body,
        out_shape=jax.ShapeDtypeStruct((np2, outf), jnp.float32),
    )(qf, v0, v1, cnt2)


def kernel(atomic_number, edge_index, lg_edge_index, h, dnr,
           key_embedding, value_table, a, b, c, d):
    n = atomic_number.shape[0]
    e = edge_index.shape[1]
    t = lg_edge_index.shape[1]
    heads = a.shape[0]
    na = key_embedding.shape[0]
    hid = key_embedding.shape[1] // heads
    outf = value_table.shape[1] // heads

    ep = _round_up(e, _NS * 2048)
    tp = _round_up(t, _NC * _NS * 1024)
    np2 = _round_up(n + 1, _NS * 128)
    nap = _round_up(na, 8)

    an_p = jnp.pad(atomic_number.astype(jnp.int32), (0, np2 - n))
    ei2 = edge_index.astype(jnp.int32).reshape(2, e // 128, 128)
    lg2 = lg_edge_index.astype(jnp.int32).reshape(2, t // 128, 128)

    h2 = h.reshape(t // 128, 128)
    dnr2 = dnr.reshape(t // 128, 128)
    kh = key_embedding.reshape(na, hid, heads).transpose(2, 0, 1)
    scal = jnp.stack([a, b % jnp.float32(np.pi), c, d])

    sp, w, eip, lgp = _prep_tc(h2, dnr2, kh, scal, ei2, lg2, n, e, ep, tp)
    wflat = w.reshape(2 * na * na)
    src1 = eip[0].reshape(ep)
    dst1 = eip[1].reshape(ep)

    sv, cp = _phase1_sc(an_p, src1, dst1, lgp, sp, wflat,
                        np2, na, e, ep, tp)
    q_out, cnt_out = _phase2_sc(an_p, eip, sv, cp, np2, np2, nap, ep)

    v = value_table.reshape(na, outf, heads)
    v0p = jnp.pad(v[:, :, 0], ((0, nap - na), (0, 0)))
    v1p = jnp.pad(v[:, :, 1], ((0, nap - na), (0, 0)))
    cnt2 = cnt_out.reshape(np2, 1)
    out_full = _final_tc(q_out, v0p, v1p, cnt2, nap, np2)
    return out_full[:n]
